# Initial kernel scaffold; baseline (speedup 1.0000x reference)
import jax
import jax.numpy as jnp
from jax.experimental import pallas as pl


def kernel(x, Wqkv, bqkv, Wcq, bcq, Wck, bck, proj_n, proj_back_n, Wproj, bproj):
    raise NotImplementedError("write your pallas kernel here")


# speedup vs baseline: 33.5504x; 33.5504x over previous
import math

import jax
import jax.numpy as jnp
from jax.experimental import pallas as pl
from jax.experimental.pallas import tpu as pltpu

_B, _N, _C, _H = 16, 577, 768, 12
_CH = _C // _H
_RC = _CH // 2
_RN = _N // 8
_TOPK = 16
_TH = 0.02
_BUDGET = math.ceil(0.25 * _N)


def _dot(a, b):
    return jnp.dot(a.astype(jnp.bfloat16), b.astype(jnp.bfloat16),
                   preferred_element_type=jnp.float32)


def _dotg(a, b, dims):
    return jax.lax.dot_general(a.astype(jnp.bfloat16), b.astype(jnp.bfloat16),
                               dims, preferred_element_type=jnp.float32)


def _kth_largest(x, k):
    xi = jax.lax.bitcast_convert_type(x, jnp.int32)

    def body(i, p):
        cand = p | (jnp.int32(1) << (30 - i))
        cnt = jnp.sum(jnp.where(xi >= cand, 1.0, 0.0), axis=-1, keepdims=True)
        return jnp.where(cnt >= k, cand, p)

    p = jax.lax.fori_loop(0, 31, body, jnp.zeros((x.shape[0], 1), jnp.int32))
    return jax.lax.bitcast_convert_type(p, jnp.float32)


def _topk_mask(x, k):
    t = _kth_largest(x, k)
    gt = x > t
    eq = x == t
    g = jnp.sum(jnp.where(gt, 1.0, 0.0), axis=-1, keepdims=True)
    c = x.shape[-1]
    ri = jax.lax.broadcasted_iota(jnp.int32, (c, c), 0)
    ci = jax.lax.broadcasted_iota(jnp.int32, (c, c), 1)
    tri = jnp.where(ri < ci, 1.0, 0.0)
    pre = _dot(jnp.where(eq, 1.0, 0.0), tri)
    return gt | (eq & (pre < (k - g)))


def _softmax(x):
    m = jnp.max(x, axis=-1, keepdims=True)
    e = jnp.exp(x - m)
    return e / jnp.sum(e, axis=-1, keepdims=True)


def _head(q, k, v, wcq, bcq, wck, bck, pn, basis_t):
    qr = _dot(q, wcq) + bcq
    kr = _dot(k, wck) + bck
    kr2 = _dotg(kr, pn, (((0,), (0,)), ((), ())))
    cheap = _dot(qr, kr2) * (_H ** -0.5)
    coef = _softmax(cheap)
    coef = jnp.where(_topk_mask(coef, _TOPK), coef, 0.0)
    approx = _dotg(coef, basis_t, (((1,), (1,)), ((), ())))
    row = jax.lax.broadcasted_iota(jnp.int32, (q.shape[0], 1), 0)
    keep = _topk_mask(approx, _BUDGET) | (row == 0)
    logits = _dotg(q * (_CH ** -0.5), k, (((1,), (1,)), ((), ())))
    logits = jnp.where(keep, logits, -jnp.inf)
    w = _softmax(logits)
    return _dot(w, v)


def _attn_kernel(q_ref, k_ref, v_ref, wcq_ref, bcq_ref, wck_ref, bck_ref,
                 pn_ref, pbn_ref, o_ref):
    pn = pn_ref[...]
    ab = jnp.abs(pbn_ref[...])
    basis_t = jnp.where(ab > _TH, ab, 0.0)
    q2, k2, v2 = q_ref[0], k_ref[0], v_ref[0]
    for j in range(2):
        sl = slice(j * _CH, (j + 1) * _CH)
        o = _head(q2[:, sl], k2[:, sl], v2[:, sl],
                  wcq_ref[...], bcq_ref[...], wck_ref[...], bck_ref[...],
                  pn, basis_t)
        o_ref[0, :, sl] = o


def _mm_bias_kernel(x_ref, w_ref, b_ref, o_ref):
    o_ref[0] = _dot(x_ref[0], w_ref[...]) + b_ref[...]


def kernel(x, Wqkv, bqkv, Wcq, bcq, Wck, bck, proj_n, proj_back_n, Wproj,
           bproj):
    qkv = pl.pallas_call(
        _mm_bias_kernel,
        grid=(_B, 3),
        in_specs=[
            pl.BlockSpec((1, _N, _C), lambda b, j: (b, 0, 0)),
            pl.BlockSpec((_C, _C), lambda b, j: (0, j)),
            pl.BlockSpec((1, _C), lambda b, j: (0, j)),
        ],
        out_specs=pl.BlockSpec((1, _N, _C), lambda b, j: (b, 0, j)),
        out_shape=jax.ShapeDtypeStruct((_B, _N, 3 * _C), jnp.float32),
        compiler_params=pltpu.CompilerParams(
            dimension_semantics=("parallel", "arbitrary")),
    )(x, Wqkv, bqkv.reshape(1, -1))

    hp = 2 * _CH
    heads = pl.pallas_call(
        _attn_kernel,
        grid=(_B, _H // 2),
        in_specs=[
            pl.BlockSpec((1, _N, hp), lambda b, h: (b, 0, h)),
            pl.BlockSpec((1, _N, hp), lambda b, h: (b, 0, _H // 2 + h)),
            pl.BlockSpec((1, _N, hp), lambda b, h: (b, 0, _H + h)),
            pl.BlockSpec((_CH, _RC), lambda b, h: (0, 0)),
            pl.BlockSpec((1, _RC), lambda b, h: (0, 0)),
            pl.BlockSpec((_CH, _RC), lambda b, h: (0, 0)),
            pl.BlockSpec((1, _RC), lambda b, h: (0, 0)),
            pl.BlockSpec((_N, _RN), lambda b, h: (0, 0)),
            pl.BlockSpec((_N, _RN), lambda b, h: (0, 0)),
        ],
        out_specs=pl.BlockSpec((1, _N, hp), lambda b, h: (b, 0, h)),
        out_shape=jax.ShapeDtypeStruct((_B, _N, _C), jnp.float32),
        compiler_params=pltpu.CompilerParams(
            dimension_semantics=("parallel", "parallel")),
    )(qkv, qkv, qkv, Wcq, bcq.reshape(1, -1), Wck, bck.reshape(1, -1),
      proj_n, proj_back_n)

    out = pl.pallas_call(
        _mm_bias_kernel,
        grid=(_B, 1),
        in_specs=[
            pl.BlockSpec((1, _N, _C), lambda b, j: (b, 0, 0)),
            pl.BlockSpec((_C, _C), lambda b, j: (0, 0)),
            pl.BlockSpec((1, _C), lambda b, j: (0, 0)),
        ],
        out_specs=pl.BlockSpec((1, _N, _C), lambda b, j: (b, 0, 0)),
        out_shape=jax.ShapeDtypeStruct((_B, _N, _C), jnp.float32),
        compiler_params=pltpu.CompilerParams(
            dimension_semantics=("parallel", "arbitrary")),
    )(heads, Wproj, bproj.reshape(1, -1))
    return out


# speedup vs baseline: 35.7088x; 1.0643x over previous
import math

import jax
import jax.numpy as jnp
from jax.experimental import pallas as pl
from jax.experimental.pallas import tpu as pltpu

_B, _N, _C, _H = 16, 577, 768, 12
_CH = _C // _H
_RC = _CH // 2
_RN = _N // 8
_TOPK = 16
_TH = 0.02
_BUDGET = math.ceil(0.25 * _N)

_BF = jnp.bfloat16
_F32 = jnp.float32


def _dot(a, b):
    return jnp.dot(a.astype(_BF), b.astype(_BF), preferred_element_type=_F32)


def _dotg(a, b, dims):
    return jax.lax.dot_general(a.astype(_BF), b.astype(_BF), dims,
                               preferred_element_type=_F32)


def _kth_largest(x, k, start_bit=29, p0_bits=0):
    xi = jax.lax.bitcast_convert_type(x, jnp.int32)

    def body(i, p):
        cand = p | (jnp.int32(1) << (start_bit - i))
        cnt = jnp.sum(jnp.where(xi >= cand, 1.0, 0.0), axis=-1, keepdims=True)
        return jnp.where(cnt >= k, cand, p)

    p0 = jnp.full((x.shape[0], 1), p0_bits, jnp.int32)
    p = jax.lax.fori_loop(0, start_bit + 1, body, p0)
    return jax.lax.bitcast_convert_type(p, jnp.float32)


def _topk_mask(x, k, tri_b, start_bit=29, p0_bits=0):
    t = _kth_largest(x, k, start_bit, p0_bits)
    gt = x > t
    eq = x == t
    g = jnp.sum(jnp.where(gt, 1.0, 0.0), axis=-1, keepdims=True)
    eq01 = jnp.where(eq, 1.0, 0.0).astype(_BF)
    pre = jnp.dot(eq01, tri_b, preferred_element_type=_F32)
    return gt | (eq & (pre < (k - g)))


def _softmax(x):
    m = jnp.max(x, axis=-1, keepdims=True)
    e = jnp.exp(x - m)
    return e / jnp.sum(e, axis=-1, keepdims=True)


def _tri(n):
    ri = jax.lax.broadcasted_iota(jnp.int32, (n, n), 0)
    ci = jax.lax.broadcasted_iota(jnp.int32, (n, n), 1)
    return jnp.where(ri < ci, 1.0, 0.0).astype(_BF)


def _head(qb, kb, vb, wcq, bcq, wck, bck, pn, basisb, tri_n, tri_c):
    qr = _dot(qb, wcq) + bcq
    kr = _dot(kb, wck) + bck
    kr2 = _dotg(kr, pn, (((0,), (0,)), ((), ())))
    cheap = _dot(qr, kr2) * (_H ** -0.5)
    coef = _softmax(cheap)
    sel = _topk_mask(coef, _TOPK, tri_c, start_bit=26, p0_bits=0x38000000)
    coefs = jnp.where(sel, coef, 0.0)
    approx = _dotg(coefs, basisb, (((1,), (1,)), ((), ())))
    row = jax.lax.broadcasted_iota(jnp.int32, (qb.shape[0], 1), 0)
    keep = _topk_mask(approx, _BUDGET, tri_n) | (row == 0)
    logits = _dotg(qb, kb, (((1,), (1,)), ((), ()))) * (_CH ** -0.5)
    logits = jnp.where(keep, logits, -jnp.inf)
    w = _softmax(logits)
    return _dot(w, vb)


def _attn_kernel(q_ref, k_ref, v_ref, wcq_ref, bcq_ref, wck_ref, bck_ref,
                 pn_ref, pbn_ref, o_ref):
    pn = pn_ref[...]
    ab = jnp.abs(pbn_ref[...])
    basisb = jnp.where(ab > _TH, ab, 0.0).astype(_BF)
    tri_n = _tri(_N)
    tri_c = _tri(_RN)
    q2, k2, v2 = q_ref[0], k_ref[0], v_ref[0]
    for j in range(2):
        sl = slice(j * _CH, (j + 1) * _CH)
        o = _head(q2[:, sl], k2[:, sl], v2[:, sl],
                  wcq_ref[...], bcq_ref[...], wck_ref[...], bck_ref[...],
                  pn, basisb, tri_n, tri_c)
        o_ref[0, :, sl] = o.astype(_BF)


def _mm_bias_kernel(x_ref, w_ref, b_ref, o_ref):
    o = _dot(x_ref[0], w_ref[...]) + b_ref[...]
    o_ref[0] = o.astype(o_ref.dtype)


def kernel(x, Wqkv, bqkv, Wcq, bcq, Wck, bck, proj_n, proj_back_n, Wproj,
           bproj):
    qkv = pl.pallas_call(
        _mm_bias_kernel,
        grid=(_B, 3),
        in_specs=[
            pl.BlockSpec((1, _N, _C), lambda b, j: (b, 0, 0)),
            pl.BlockSpec((_C, _C), lambda b, j: (0, j)),
            pl.BlockSpec((1, _C), lambda b, j: (0, j)),
        ],
        out_specs=pl.BlockSpec((1, _N, _C), lambda b, j: (b, 0, j)),
        out_shape=jax.ShapeDtypeStruct((_B, _N, 3 * _C), _BF),
        compiler_params=pltpu.CompilerParams(
            dimension_semantics=("parallel", "arbitrary")),
    )(x.astype(_BF), Wqkv.astype(_BF), bqkv.reshape(1, -1))

    hp = 2 * _CH
    heads = pl.pallas_call(
        _attn_kernel,
        grid=(_B, _H // 2),
        in_specs=[
            pl.BlockSpec((1, _N, hp), lambda b, h: (b, 0, h)),
            pl.BlockSpec((1, _N, hp), lambda b, h: (b, 0, _H // 2 + h)),
            pl.BlockSpec((1, _N, hp), lambda b, h: (b, 0, _H + h)),
            pl.BlockSpec((_CH, _RC), lambda b, h: (0, 0)),
            pl.BlockSpec((1, _RC), lambda b, h: (0, 0)),
            pl.BlockSpec((_CH, _RC), lambda b, h: (0, 0)),
            pl.BlockSpec((1, _RC), lambda b, h: (0, 0)),
            pl.BlockSpec((_N, _RN), lambda b, h: (0, 0)),
            pl.BlockSpec((_N, _RN), lambda b, h: (0, 0)),
        ],
        out_specs=pl.BlockSpec((1, _N, hp), lambda b, h: (b, 0, h)),
        out_shape=jax.ShapeDtypeStruct((_B, _N, _C), _BF),
        compiler_params=pltpu.CompilerParams(
            dimension_semantics=("parallel", "parallel")),
    )(qkv, qkv, qkv, Wcq.astype(_BF), bcq.reshape(1, -1),
      Wck.astype(_BF), bck.reshape(1, -1), proj_n.astype(_BF), proj_back_n)

    out = pl.pallas_call(
        _mm_bias_kernel,
        grid=(_B, 1),
        in_specs=[
            pl.BlockSpec((1, _N, _C), lambda b, j: (b, 0, 0)),
            pl.BlockSpec((_C, _C), lambda b, j: (0, 0)),
            pl.BlockSpec((1, _C), lambda b, j: (0, 0)),
        ],
        out_specs=pl.BlockSpec((1, _N, _C), lambda b, j: (b, 0, 0)),
        out_shape=jax.ShapeDtypeStruct((_B, _N, _C), _F32),
        compiler_params=pltpu.CompilerParams(
            dimension_semantics=("parallel", "arbitrary")),
    )(heads, Wproj.astype(_BF), bproj.reshape(1, -1))
    return out


# speedup vs baseline: 60.0915x; 1.6828x over previous
import math

import jax
import jax.numpy as jnp
from jax.experimental import pallas as pl
from jax.experimental.pallas import tpu as pltpu

_B, _N, _C, _H = 16, 577, 768, 12
_CH = _C // _H
_RC = _CH // 2
_RN = _N // 8
_TOPK = 16
_TH = 0.02
_BUDGET = math.ceil(0.25 * _N)

_BF = jnp.bfloat16
_F32 = jnp.float32


def _dot(a, b):
    return jnp.dot(a.astype(_BF), b.astype(_BF), preferred_element_type=_F32)


def _dotg(a, b, dims):
    return jax.lax.dot_general(a.astype(_BF), b.astype(_BF), dims,
                               preferred_element_type=_F32)


def _kth_largest_t(x, k, start_bit=29, p0_bits=0):
    xi = jax.lax.bitcast_convert_type(x, jnp.int32)

    def body(i, p):
        cand = p | (jnp.int32(1) << (start_bit - i))
        cnt = jnp.sum(jnp.where(xi >= cand, 1.0, 0.0), axis=0, keepdims=True)
        return jnp.where(cnt >= k, cand, p)

    p0 = jnp.full((1, x.shape[1]), p0_bits, jnp.int32)
    p = jax.lax.fori_loop(0, start_bit + 1, body, p0)
    return jax.lax.bitcast_convert_type(p, jnp.float32)


def _topk_mask_t(x, k, ltri_b, start_bit=29, p0_bits=0):
    t = _kth_largest_t(x, k, start_bit, p0_bits)
    gt = x > t
    eq = x == t
    g = jnp.sum(jnp.where(gt, 1.0, 0.0), axis=0, keepdims=True)
    eq01 = jnp.where(eq, 1.0, 0.0).astype(_BF)
    pre = jnp.dot(ltri_b, eq01, preferred_element_type=_F32)
    return gt | (eq & (pre < (k - g)))


def _softmax_t(x):
    m = jnp.max(x, axis=0, keepdims=True)
    e = jnp.exp(x - m)
    return e / jnp.sum(e, axis=0, keepdims=True)


def _ltri(n):
    ri = jax.lax.broadcasted_iota(jnp.int32, (n, n), 0)
    ci = jax.lax.broadcasted_iota(jnp.int32, (n, n), 1)
    return jnp.where(ri > ci, 1.0, 0.0).astype(_BF)


def _head(qb, kb, vb, wcq, bcq, wck, bck, pn, basisb, ltri_n, ltri_c):
    qr = _dot(qb, wcq) + bcq
    kr = _dot(kb, wck) + bck
    kr2 = _dotg(kr, pn, (((0,), (0,)), ((), ())))
    cheap_t = _dotg(kr2, qr, (((0,), (1,)), ((), ()))) * (_H ** -0.5)
    coef_t = _softmax_t(cheap_t)
    sel_t = _topk_mask_t(coef_t, _TOPK, ltri_c, start_bit=26,
                         p0_bits=0x38000000)
    coefs_t = jnp.where(sel_t, coef_t, 0.0)
    approx_t = _dot(basisb, coefs_t)
    col = jax.lax.broadcasted_iota(jnp.int32, (1, qb.shape[0]), 1)
    keep_t = _topk_mask_t(approx_t, _BUDGET, ltri_n) | (col == 0)
    logits_t = _dotg(kb, qb, (((1,), (1,)), ((), ()))) * (_CH ** -0.5)
    logits_t = jnp.where(keep_t, logits_t, -jnp.inf)
    w_t = _softmax_t(logits_t)
    return _dotg(w_t, vb, (((0,), (0,)), ((), ())))


def _attn_kernel(q_ref, k_ref, v_ref, wcq_ref, bcq_ref, wck_ref, bck_ref,
                 pn_ref, pbn_ref, o_ref):
    pn = pn_ref[...]
    ab = jnp.abs(pbn_ref[...])
    basisb = jnp.where(ab > _TH, ab, 0.0).astype(_BF)
    ltri_n = _ltri(_N)
    ltri_c = _ltri(_RN)
    q2, k2, v2 = q_ref[0], k_ref[0], v_ref[0]
    for j in range(2):
        sl = slice(j * _CH, (j + 1) * _CH)
        o = _head(q2[:, sl], k2[:, sl], v2[:, sl],
                  wcq_ref[...], bcq_ref[...], wck_ref[...], bck_ref[...],
                  pn, basisb, ltri_n, ltri_c)
        o_ref[0, :, sl] = o.astype(_BF)


def _mm_bias_kernel(x_ref, w_ref, b_ref, o_ref):
    o = _dot(x_ref[0], w_ref[...]) + b_ref[...]
    o_ref[0] = o.astype(o_ref.dtype)


def kernel(x, Wqkv, bqkv, Wcq, bcq, Wck, bck, proj_n, proj_back_n, Wproj,
           bproj):
    qkv = pl.pallas_call(
        _mm_bias_kernel,
        grid=(_B, 3),
        in_specs=[
            pl.BlockSpec((1, _N, _C), lambda b, j: (b, 0, 0)),
            pl.BlockSpec((_C, _C), lambda b, j: (0, j)),
            pl.BlockSpec((1, _C), lambda b, j: (0, j)),
        ],
        out_specs=pl.BlockSpec((1, _N, _C), lambda b, j: (b, 0, j)),
        out_shape=jax.ShapeDtypeStruct((_B, _N, 3 * _C), _BF),
        compiler_params=pltpu.CompilerParams(
            dimension_semantics=("parallel", "arbitrary")),
    )(x.astype(_BF), Wqkv.astype(_BF), bqkv.reshape(1, -1))

    hp = 2 * _CH
    heads = pl.pallas_call(
        _attn_kernel,
        grid=(_B, _H // 2),
        in_specs=[
            pl.BlockSpec((1, _N, hp), lambda b, h: (b, 0, h)),
            pl.BlockSpec((1, _N, hp), lambda b, h: (b, 0, _H // 2 + h)),
            pl.BlockSpec((1, _N, hp), lambda b, h: (b, 0, _H + h)),
            pl.BlockSpec((_CH, _RC), lambda b, h: (0, 0)),
            pl.BlockSpec((1, _RC), lambda b, h: (0, 0)),
            pl.BlockSpec((_CH, _RC), lambda b, h: (0, 0)),
            pl.BlockSpec((1, _RC), lambda b, h: (0, 0)),
            pl.BlockSpec((_N, _RN), lambda b, h: (0, 0)),
            pl.BlockSpec((_N, _RN), lambda b, h: (0, 0)),
        ],
        out_specs=pl.BlockSpec((1, _N, hp), lambda b, h: (b, 0, h)),
        out_shape=jax.ShapeDtypeStruct((_B, _N, _C), _BF),
        compiler_params=pltpu.CompilerParams(
            dimension_semantics=("parallel", "parallel")),
    )(qkv, qkv, qkv, Wcq.astype(_BF), bcq.reshape(1, -1),
      Wck.astype(_BF), bck.reshape(1, -1), proj_n.astype(_BF), proj_back_n)

    out = pl.pallas_call(
        _mm_bias_kernel,
        grid=(_B, 1),
        in_specs=[
            pl.BlockSpec((1, _N, _C), lambda b, j: (b, 0, 0)),
            pl.BlockSpec((_C, _C), lambda b, j: (0, 0)),
            pl.BlockSpec((1, _C), lambda b, j: (0, 0)),
        ],
        out_specs=pl.BlockSpec((1, _N, _C), lambda b, j: (b, 0, 0)),
        out_shape=jax.ShapeDtypeStruct((_B, _N, _C), _F32),
        compiler_params=pltpu.CompilerParams(
            dimension_semantics=("parallel", "arbitrary")),
    )(heads, Wproj.astype(_BF), bproj.reshape(1, -1))
    return out


# speedup vs baseline: 63.2021x; 1.0518x over previous
import math

import jax
import jax.numpy as jnp
from jax.experimental import pallas as pl
from jax.experimental.pallas import tpu as pltpu

_B, _N, _C, _H = 16, 577, 768, 12
_CH = _C // _H
_RC = _CH // 2
_RN = _N // 8
_TOPK = 16
_TH = 0.02
_BUDGET = math.ceil(0.25 * _N)

_BF = jnp.bfloat16
_F32 = jnp.float32


def _dot(a, b):
    return jnp.dot(a.astype(_BF), b.astype(_BF), preferred_element_type=_F32)


def _dotg(a, b, dims):
    return jax.lax.dot_general(a.astype(_BF), b.astype(_BF), dims,
                               preferred_element_type=_F32)


def _kth_largest_t(x, k, start_bit=29, p0_bits=0):
    xi = jax.lax.bitcast_convert_type(x, jnp.int32)

    def body(i, p):
        cand = p | (jnp.int32(1) << (start_bit - i))
        cnt = jnp.sum(jnp.where(xi >= cand, 1.0, 0.0), axis=0, keepdims=True)
        return jnp.where(cnt >= k, cand, p)

    p0 = jnp.full((1, x.shape[1]), p0_bits, jnp.int32)
    p = jax.lax.fori_loop(0, start_bit + 1, body, p0)
    return jax.lax.bitcast_convert_type(p, jnp.float32)


def _kth_largest_t2(xa, xb, k, start_bit=29, p0_bits=0):
    xia = jax.lax.bitcast_convert_type(xa, jnp.int32)
    xib = jax.lax.bitcast_convert_type(xb, jnp.int32)

    def body(i, ps):
        pa, pb = ps
        bit = jnp.int32(1) << (start_bit - i)
        ca = pa | bit
        cb = pb | bit
        cnta = jnp.sum(jnp.where(xia >= ca, 1.0, 0.0), axis=0, keepdims=True)
        cntb = jnp.sum(jnp.where(xib >= cb, 1.0, 0.0), axis=0, keepdims=True)
        return (jnp.where(cnta >= k, ca, pa), jnp.where(cntb >= k, cb, pb))

    p0 = jnp.full((1, xa.shape[1]), p0_bits, jnp.int32)
    pa, pb = jax.lax.fori_loop(0, start_bit + 1, body, (p0, p0))
    return (jax.lax.bitcast_convert_type(pa, jnp.float32),
            jax.lax.bitcast_convert_type(pb, jnp.float32))


def _topk_mask_t2(xa, xb, k, ltri_b, start_bit=29, p0_bits=0):
    ta, tb = _kth_largest_t2(xa, xb, k, start_bit, p0_bits)
    out = []
    for x, t in ((xa, ta), (xb, tb)):
        gt = x > t
        eq = x == t
        g = jnp.sum(jnp.where(gt, 1.0, 0.0), axis=0, keepdims=True)
        eq01 = jnp.where(eq, 1.0, 0.0).astype(_BF)
        pre = jnp.dot(ltri_b, eq01, preferred_element_type=_F32)
        out.append(gt | (eq & (pre < (k - g))))
    return out


def _topk_mask_t(x, k, ltri_b, start_bit=29, p0_bits=0):
    t = _kth_largest_t(x, k, start_bit, p0_bits)
    gt = x > t
    eq = x == t
    g = jnp.sum(jnp.where(gt, 1.0, 0.0), axis=0, keepdims=True)
    eq01 = jnp.where(eq, 1.0, 0.0).astype(_BF)
    pre = jnp.dot(ltri_b, eq01, preferred_element_type=_F32)
    return gt | (eq & (pre < (k - g)))


def _softmax_t(x):
    m = jnp.max(x, axis=0, keepdims=True)
    e = jnp.exp(x - m)
    return e / jnp.sum(e, axis=0, keepdims=True)


def _ltri(n):
    ri = jax.lax.broadcasted_iota(jnp.int32, (n, n), 0)
    ci = jax.lax.broadcasted_iota(jnp.int32, (n, n), 1)
    return jnp.where(ri > ci, 1.0, 0.0).astype(_BF)


def _head_scores(qb, kb, wcq, bcq, wck, bck, pn):
    qr = _dot(qb, wcq) + bcq
    kr = _dot(kb, wck) + bck
    kr2 = _dotg(kr, pn, (((0,), (0,)), ((), ())))
    cheap_t = _dotg(kr2, qr, (((0,), (1,)), ((), ()))) * (_H ** -0.5)
    return _softmax_t(cheap_t)


def _attn_kernel(q_ref, k_ref, v_ref, wcq_ref, bcq_ref, wck_ref, bck_ref,
                 pn_ref, pbn_ref, o_ref):
    pn = pn_ref[...]
    ab = jnp.abs(pbn_ref[...])
    basisb = jnp.where(ab > _TH, ab, 0.0).astype(_BF)
    ltri_n = _ltri(_N)
    ltri_c = _ltri(_RN)
    qs, ks, vs = [], [], []
    for j in range(2):
        sl = slice(j * _CH, (j + 1) * _CH)
        qs.append(q_ref[0][:, sl])
        ks.append(k_ref[0][:, sl])
        vs.append(v_ref[0][:, sl])
    coef_t = [_head_scores(qs[j], ks[j], wcq_ref[...], bcq_ref[...],
                           wck_ref[...], bck_ref[...], pn) for j in range(2)]
    sels = _topk_mask_t2(coef_t[0], coef_t[1], _TOPK, ltri_c,
                         start_bit=26, p0_bits=0x38000000)
    approx_t = [_dot(basisb, jnp.where(sels[j], coef_t[j], 0.0))
                for j in range(2)]
    col = jax.lax.broadcasted_iota(jnp.int32, (1, _N), 1)
    keeps = _topk_mask_t2(approx_t[0], approx_t[1], _BUDGET, ltri_n)
    for j in range(2):
        keep_t = keeps[j] | (col == 0)
        logits_t = _dotg(ks[j], qs[j], (((1,), (1,)), ((), ()))) \
            * (_CH ** -0.5)
        logits_t = jnp.where(keep_t, logits_t, -jnp.inf)
        w_t = _softmax_t(logits_t)
        o = _dotg(w_t, vs[j], (((0,), (0,)), ((), ())))
        o_ref[0, :, slice(j * _CH, (j + 1) * _CH)] = o.astype(_BF)


def _mm_bias_kernel(x_ref, w_ref, b_ref, o_ref):
    o = _dot(x_ref[0], w_ref[...]) + b_ref[...]
    o_ref[0] = o.astype(o_ref.dtype)


def kernel(x, Wqkv, bqkv, Wcq, bcq, Wck, bck, proj_n, proj_back_n, Wproj,
           bproj):
    qkv = pl.pallas_call(
        _mm_bias_kernel,
        grid=(_B, 3),
        in_specs=[
            pl.BlockSpec((1, _N, _C), lambda b, j: (b, 0, 0)),
            pl.BlockSpec((_C, _C), lambda b, j: (0, j)),
            pl.BlockSpec((1, _C), lambda b, j: (0, j)),
        ],
        out_specs=pl.BlockSpec((1, _N, _C), lambda b, j: (b, 0, j)),
        out_shape=jax.ShapeDtypeStruct((_B, _N, 3 * _C), _BF),
        compiler_params=pltpu.CompilerParams(
            dimension_semantics=("parallel", "arbitrary")),
    )(x.astype(_BF), Wqkv.astype(_BF), bqkv.reshape(1, -1))

    hp = 2 * _CH
    heads = pl.pallas_call(
        _attn_kernel,
        grid=(_B, _H // 2),
        in_specs=[
            pl.BlockSpec((1, _N, hp), lambda b, h: (b, 0, h)),
            pl.BlockSpec((1, _N, hp), lambda b, h: (b, 0, _H // 2 + h)),
            pl.BlockSpec((1, _N, hp), lambda b, h: (b, 0, _H + h)),
            pl.BlockSpec((_CH, _RC), lambda b, h: (0, 0)),
            pl.BlockSpec((1, _RC), lambda b, h: (0, 0)),
            pl.BlockSpec((_CH, _RC), lambda b, h: (0, 0)),
            pl.BlockSpec((1, _RC), lambda b, h: (0, 0)),
            pl.BlockSpec((_N, _RN), lambda b, h: (0, 0)),
            pl.BlockSpec((_N, _RN), lambda b, h: (0, 0)),
        ],
        out_specs=pl.BlockSpec((1, _N, hp), lambda b, h: (b, 0, h)),
        out_shape=jax.ShapeDtypeStruct((_B, _N, _C), _BF),
        compiler_params=pltpu.CompilerParams(
            dimension_semantics=("parallel", "parallel")),
    )(qkv, qkv, qkv, Wcq.astype(_BF), bcq.reshape(1, -1),
      Wck.astype(_BF), bck.reshape(1, -1), proj_n.astype(_BF), proj_back_n)

    out = pl.pallas_call(
        _mm_bias_kernel,
        grid=(_B, 1),
        in_specs=[
            pl.BlockSpec((1, _N, _C), lambda b, j: (b, 0, 0)),
            pl.BlockSpec((_C, _C), lambda b, j: (0, 0)),
            pl.BlockSpec((1, _C), lambda b, j: (0, 0)),
        ],
        out_specs=pl.BlockSpec((1, _N, _C), lambda b, j: (b, 0, 0)),
        out_shape=jax.ShapeDtypeStruct((_B, _N, _C), _F32),
        compiler_params=pltpu.CompilerParams(
            dimension_semantics=("parallel", "arbitrary")),
    )(heads, Wproj.astype(_BF), bproj.reshape(1, -1))
    return out
